# baseline (speedup 1.0000x reference)
import functools

import jax
import jax.numpy as jnp
from jax import lax
from jax.experimental import pallas as pl
from jax.experimental.pallas import tpu as pltpu
from jax.experimental.pallas import tpu_sc as plsc

K = 32
LANES = 16


def _encode_body(n_blocks, xb_ref, wb_ref, bp_ref, hs_ref, cnt_ref, c16_ref,
                 h_acc):
    j = pl.program_id(1)
    bn = wb_ref.shape[1]
    xb = (xb_ref[...] - bp_ref[...]).astype(jnp.bfloat16)
    hblk = jnp.dot(xb, wb_ref[...], preferred_element_type=jnp.float32)
    hblk = jnp.maximum(hblk, 0.0)
    h_acc[:, pl.ds(j * bn, bn)] = hblk

    @pl.when(j == n_blocks - 1)
    def _():
        h = h_acc[...]
        br, n = h.shape
        m = jnp.max(h.reshape(br, K, n // K), axis=2)
        lo = lax.bitcast_convert_type(jnp.min(m, axis=1, keepdims=True),
                                      jnp.int32)
        hi = lax.bitcast_convert_type(jnp.max(m, axis=1, keepdims=True),
                                      jnp.int32) + 1

        def cond(carry):
            lo, hi = carry
            return jnp.any(hi - lo > 1)

        def body(carry):
            lo, hi = carry
            mid = lo + lax.div(hi - lo, 2)
            midf = lax.bitcast_convert_type(mid, jnp.float32)
            cnt = jnp.sum((h >= midf).astype(jnp.float32), axis=1,
                          keepdims=True)
            ge = cnt >= float(K)
            return (jnp.where(ge, mid, lo), jnp.where(ge, hi, mid))

        lo, hi = lax.while_loop(cond, body, (lo, hi))
        t = lax.bitcast_convert_type(lo, jnp.float32)
        hs = jnp.where(h >= t, h, 0.0)
        hs_ref[...] = hs
        pos = (hs > 0.0).astype(jnp.float32)
        cnt_ref[...] = jnp.sum(pos, axis=1, keepdims=True)
        c16_ref[...] = jnp.sum(pos.reshape(br, n // 128, 128), axis=2)


def _splat(v, k):
    idx = jnp.full((LANES, 1), k, jnp.int32)
    dn = lax.GatherDimensionNumbers(offset_dims=(), collapsed_slice_dims=(0,),
                                    start_index_map=(0,))
    return lax.gather(v, idx, dn, (1,),
                      mode=lax.GatherScatterMode.PROMISE_IN_BOUNDS)


def _sc_decode_body(B, N, D, hsr, c16, wd, x, bp, xhat, res,
                    c16row, vidbuf, hrow, idxbuf, valbuf, wrows,
                    xrow, xhrow, bprow, resbuf, sem):
    nw = 32
    bpw = B // nw
    ng = N // 128 // LANES
    nchunk = 4
    cw = D // nchunk
    wid = lax.axis_index("s") * 2 + lax.axis_index("c")
    base = wid * bpw
    iota = lax.iota(jnp.int32, LANES)
    zeros_i = jnp.zeros((LANES,), jnp.int32)
    zeros_f = jnp.zeros((LANES,), jnp.float32)

    pltpu.sync_copy(bp, bprow)
    vidbuf[pl.ds(0, LANES)] = zeros_i
    vidbuf[pl.ds(LANES, LANES)] = zeros_i

    def row_body(r, carry):
        row = base + r
        pltpu.sync_copy(c16.at[row], c16row)
        pltpu.sync_copy(x.at[row], xrow)
        pltpu.sync_copy(hsr.at[row], hrow)

        def scan_body(g, nhit):
            v = c16row[pl.ds(g * LANES, LANES)]
            m = v > 0.0
            ranks = plsc.cumsum(jnp.where(m, 1, 0)) - 1
            plsc.store_scatter(vidbuf, [nhit + ranks], g * LANES + iota,
                               mask=m)
            return nhit + plsc.all_reduce_population_count(m)

        nhit = lax.fori_loop(0, ng, scan_body, zeros_i)

        v0 = vidbuf[pl.ds(0, LANES)]
        v1 = vidbuf[pl.ds(LANES, LANES)]

        idxbuf[pl.ds(0, LANES)] = zeros_i
        idxbuf[pl.ds(LANES, LANES)] = zeros_i
        valbuf[pl.ds(0, LANES)] = zeros_f
        valbuf[pl.ds(LANES, LANES)] = zeros_f
        nsel = zeros_i
        for k in range(K):
            gidk = _splat(v0 if k < LANES else v1, k % LANES)
            for j in range(128 // LANES):
                lidx = gidk * 128 + j * LANES + iota
                hv = plsc.load_gather(hrow, [lidx])
                act = (hv > 0.0) & (k < nhit)
                ranks = plsc.cumsum(jnp.where(act, 1, 0)) - 1
                tgt = nsel + ranks
                plsc.store_scatter(valbuf, [tgt], hv, mask=act)
                plsc.store_scatter(idxbuf, [tgt], lidx, mask=act)
                nsel = nsel + plsc.all_reduce_population_count(act)

        pltpu.async_copy(wd.at[idxbuf.at[pl.ds(0, K)]], wrows, sem).wait()

        w0 = valbuf[pl.ds(0, LANES)]
        w1 = valbuf[pl.ds(LANES, LANES)]

        def chunk_body(c, racc):
            accs = [zeros_f] * (cw // LANES)
            for k in range(K):
                vs = _splat(w0 if k < LANES else w1, k % LANES)
                for i in range(cw // LANES):
                    accs[i] = accs[i] + vs * wrows[k,
                                                   pl.ds(c * cw + i * LANES,
                                                         LANES)]
            for i in range(cw // LANES):
                off = c * cw + i * LANES
                xh = accs[i] + bprow[pl.ds(off, LANES)]
                xhrow[pl.ds(off, LANES)] = xh
                d = xrow[pl.ds(off, LANES)] - xh
                racc = racc + d * d
            return racc

        racc = lax.fori_loop(0, nchunk, chunk_body, zeros_f)
        pltpu.sync_copy(xhrow, xhat.at[row])
        rsum = jnp.sum(racc)
        plsc.store_scatter(resbuf, [jnp.full((LANES,), r, jnp.int32)],
                           jnp.full((LANES,), rsum, jnp.float32),
                           mask=iota == 0)
        return carry

    lax.fori_loop(0, bpw, row_body, 0)
    pltpu.sync_copy(resbuf, res.at[pl.ds(base, bpw)])


def kernel(x, W_enc, W_dec, b_pre):
    B, D = x.shape
    N = W_enc.shape[1]
    bp2 = b_pre.reshape(1, D)

    BR = min(128, B)
    BN = min(512, N)
    rb, nb = B // BR, N // BN
    we16 = W_enc.astype(jnp.bfloat16)
    hs, cnt, c16 = pl.pallas_call(
        functools.partial(_encode_body, nb),
        grid=(rb, nb),
        in_specs=[
            pl.BlockSpec((BR, D), lambda r, n: (r, 0)),
            pl.BlockSpec((D, BN), lambda r, n: (0, n)),
            pl.BlockSpec((1, D), lambda r, n: (0, 0)),
        ],
        out_specs=[
            pl.BlockSpec((BR, N), lambda r, n: (r, 0)),
            pl.BlockSpec((BR, 1), lambda r, n: (r, 0)),
            pl.BlockSpec((BR, N // 128), lambda r, n: (r, 0)),
        ],
        out_shape=[
            jax.ShapeDtypeStruct((B, N), jnp.float32),
            jax.ShapeDtypeStruct((B, 1), jnp.float32),
            jax.ShapeDtypeStruct((B, N // 128), jnp.float32),
        ],
        scratch_shapes=[pltpu.VMEM((BR, N), jnp.float32)],
        compiler_params=pltpu.CompilerParams(
            dimension_semantics=("parallel", "arbitrary")),
    )(x, we16, bp2)

    mesh = plsc.VectorSubcoreMesh(core_axis_name="c", subcore_axis_name="s")
    xhat, res = pl.kernel(
        functools.partial(_sc_decode_body, B, N, D),
        out_type=[
            jax.ShapeDtypeStruct((B, D), jnp.float32),
            jax.ShapeDtypeStruct((B,), jnp.float32),
        ],
        mesh=mesh,
        scratch_types=[
            pltpu.VMEM((N // 128,), jnp.float32),
            pltpu.VMEM((N // 128 + LANES,), jnp.int32),
            pltpu.VMEM((N,), jnp.float32),
            pltpu.VMEM((K * LANES + LANES,), jnp.int32),
            pltpu.VMEM((K * LANES + LANES,), jnp.float32),
            pltpu.VMEM((K, D), jnp.float32),
            pltpu.VMEM((D,), jnp.float32),
            pltpu.VMEM((D,), jnp.float32),
            pltpu.VMEM((D,), jnp.float32),
            pltpu.VMEM((B // 32,), jnp.float32),
            pltpu.SemaphoreType.DMA,
        ],
        compiler_params=pltpu.CompilerParams(needs_layout_passes=False),
    )(hs, c16, W_dec, x, b_pre)

    recon_loss = jnp.sum(res) / (B * D)
    l0 = jnp.sum(cnt) / B
    return (xhat, hs, recon_loss, l0)


# speedup vs baseline: 2.1736x; 2.1736x over previous
import functools

import jax
import jax.numpy as jnp
from jax import lax
from jax.experimental import pallas as pl
from jax.experimental.pallas import tpu as pltpu

K = 32


def _encode_body(n_blocks, xb_ref, wb_ref, bp_ref, hs_ref, cnt_ref, h_acc):
    j = pl.program_id(1)
    bn = wb_ref.shape[1]
    xb = (xb_ref[...] - bp_ref[...]).astype(jnp.bfloat16)
    hblk = jnp.dot(xb, wb_ref[...], preferred_element_type=jnp.float32)
    hblk = jnp.maximum(hblk, 0.0)
    h_acc[:, pl.ds(j * bn, bn)] = hblk

    @pl.when(j == n_blocks - 1)
    def _():
        h = h_acc[...]
        br, n = h.shape
        m = jnp.max(h.reshape(br, K, n // K), axis=2)
        lo = lax.bitcast_convert_type(jnp.min(m, axis=1, keepdims=True),
                                      jnp.int32)
        hi = lax.bitcast_convert_type(jnp.max(m, axis=1, keepdims=True),
                                      jnp.int32) + 1

        def cond(carry):
            lo, hi = carry
            return jnp.any(hi - lo > 1)

        ones = jnp.ones((n, 8), jnp.bfloat16)

        def body(carry):
            lo, hi = carry
            mid = lo + lax.div(hi - lo, 2)
            midf = lax.bitcast_convert_type(mid, jnp.float32)
            m01 = (h >= midf).astype(jnp.bfloat16)
            cnt = jnp.dot(m01, ones,
                          preferred_element_type=jnp.float32)[:, :1]
            ge = cnt >= float(K)
            return (jnp.where(ge, mid, lo), jnp.where(ge, hi, mid))

        lo, hi = lax.while_loop(cond, body, (lo, hi))
        t = lax.bitcast_convert_type(lo, jnp.float32)
        hs = jnp.where(h >= t, h, 0.0)
        hs_ref[...] = hs
        cnt_ref[...] = jnp.sum((hs > 0.0).astype(jnp.float32), axis=1,
                               keepdims=True)


def _decode_body(k_blocks, hs_ref, wd_ref, x_ref, bp_ref, xhat_ref, res_ref,
                 acc):
    k = pl.program_id(1)

    @pl.when(k == 0)
    def _():
        acc[...] = jnp.zeros_like(acc)

    acc[...] += jnp.dot(hs_ref[...].astype(jnp.bfloat16), wd_ref[...],
                        preferred_element_type=jnp.float32)

    @pl.when(k == k_blocks - 1)
    def _():
        xh = acc[...] + bp_ref[...]
        xhat_ref[...] = xh
        d = x_ref[...] - xh
        res_ref[...] = jnp.broadcast_to(jnp.sum(d * d), (1, 1, 128))


def kernel(x, W_enc, W_dec, b_pre):
    B, D = x.shape
    N = W_enc.shape[1]
    bp2 = b_pre.reshape(1, D)

    BR = min(128, B)
    BN = min(512, N)
    rb, nb = B // BR, N // BN
    we16 = W_enc.astype(jnp.bfloat16)
    hs, cnt = pl.pallas_call(
        functools.partial(_encode_body, nb),
        grid=(rb, nb),
        in_specs=[
            pl.BlockSpec((BR, D), lambda r, n: (r, 0)),
            pl.BlockSpec((D, BN), lambda r, n: (0, n)),
            pl.BlockSpec((1, D), lambda r, n: (0, 0)),
        ],
        out_specs=[
            pl.BlockSpec((BR, N), lambda r, n: (r, 0)),
            pl.BlockSpec((BR, 1), lambda r, n: (r, 0)),
        ],
        out_shape=[
            jax.ShapeDtypeStruct((B, N), jnp.float32),
            jax.ShapeDtypeStruct((B, 1), jnp.float32),
        ],
        scratch_shapes=[pltpu.VMEM((BR, N), jnp.float32)],
        compiler_params=pltpu.CompilerParams(
            dimension_semantics=("parallel", "arbitrary")),
    )(x, we16, bp2)

    BR2 = min(512, B)
    BK = min(2048, N)
    rb2, kb = B // BR2, N // BK
    wd16 = W_dec.astype(jnp.bfloat16)
    xhat, res = pl.pallas_call(
        functools.partial(_decode_body, kb),
        grid=(rb2, kb),
        in_specs=[
            pl.BlockSpec((BR2, BK), lambda r, k: (r, k)),
            pl.BlockSpec((BK, D), lambda r, k: (k, 0)),
            pl.BlockSpec((BR2, D), lambda r, k: (r, 0)),
            pl.BlockSpec((1, D), lambda r, k: (0, 0)),
        ],
        out_specs=[
            pl.BlockSpec((BR2, D), lambda r, k: (r, 0)),
            pl.BlockSpec((1, 1, 128), lambda r, k: (r, 0, 0)),
        ],
        out_shape=[
            jax.ShapeDtypeStruct((B, D), jnp.float32),
            jax.ShapeDtypeStruct((rb2, 1, 128), jnp.float32),
        ],
        scratch_shapes=[pltpu.VMEM((BR2, D), jnp.float32)],
        compiler_params=pltpu.CompilerParams(
            dimension_semantics=("parallel", "arbitrary")),
    )(hs, wd16, x, bp2)

    recon_loss = jnp.sum(res[:, 0, 0]) / (B * D)
    l0 = jnp.sum(cnt) / B
    return (xhat, hs, recon_loss, l0)


# speedup vs baseline: 2.4463x; 1.1255x over previous
import functools

import jax
import jax.numpy as jnp
from jax import lax
from jax.experimental import pallas as pl
from jax.experimental.pallas import tpu as pltpu

K = 32


def _encode_body(n_blocks, xb_ref, wb_ref, bp_ref, hs_ref, cnt_ref, h_acc):
    j = pl.program_id(1)
    bn = wb_ref.shape[1]
    xb = (xb_ref[...] - bp_ref[...]).astype(jnp.bfloat16)
    hblk = jnp.dot(xb, wb_ref[...], preferred_element_type=jnp.float32)
    hblk = jnp.maximum(hblk, 0.0)
    h_acc[:, pl.ds(j * bn, bn)] = hblk

    @pl.when(j == n_blocks - 1)
    def _():
        h = h_acc[...]
        br, n = h.shape
        m = jnp.max(h.reshape(br, K, n // K), axis=2)
        lo = lax.bitcast_convert_type(jnp.min(m, axis=1, keepdims=True),
                                      jnp.int32)
        hi = lax.bitcast_convert_type(jnp.max(m, axis=1, keepdims=True),
                                      jnp.int32) + 1

        def cond(carry):
            lo, hi = carry
            return jnp.any(hi - lo > 1)

        def body(carry):
            lo, hi = carry
            mid = lo + lax.div(hi - lo, 2)
            midf = lax.bitcast_convert_type(mid, jnp.float32)
            cnt = jnp.sum((h >= midf).astype(jnp.float32), axis=1,
                          keepdims=True)
            ge = cnt >= float(K)
            return (jnp.where(ge, mid, lo), jnp.where(ge, hi, mid))

        lo, hi = lax.while_loop(cond, body, (lo, hi))
        t = lax.bitcast_convert_type(lo, jnp.float32)
        hs = jnp.where(h >= t, h, 0.0)
        hs_ref[...] = hs
        cnt_ref[...] = jnp.sum((hs > 0.0).astype(jnp.float32), axis=1,
                               keepdims=True)


def _decode_body(k_blocks, hs_ref, wd_ref, x_ref, bp_ref, xhat_ref, res_ref,
                 acc):
    k = pl.program_id(1)

    @pl.when(k == 0)
    def _():
        acc[...] = jnp.zeros_like(acc)

    acc[...] += jnp.dot(hs_ref[...].astype(jnp.bfloat16), wd_ref[...],
                        preferred_element_type=jnp.float32)

    @pl.when(k == k_blocks - 1)
    def _():
        xh = acc[...] + bp_ref[...]
        xhat_ref[...] = xh
        d = x_ref[...] - xh
        res_ref[...] = jnp.broadcast_to(jnp.sum(d * d), (1, 1, 128))


def kernel(x, W_enc, W_dec, b_pre):
    B, D = x.shape
    N = W_enc.shape[1]
    bp2 = b_pre.reshape(1, D)

    BR = min(128, B)
    BN = min(512, N)
    rb, nb = B // BR, N // BN
    we16 = W_enc.astype(jnp.bfloat16)
    hs, cnt = pl.pallas_call(
        functools.partial(_encode_body, nb),
        grid=(rb, nb),
        in_specs=[
            pl.BlockSpec((BR, D), lambda r, n: (r, 0)),
            pl.BlockSpec((D, BN), lambda r, n: (0, n)),
            pl.BlockSpec((1, D), lambda r, n: (0, 0)),
        ],
        out_specs=[
            pl.BlockSpec((BR, N), lambda r, n: (r, 0)),
            pl.BlockSpec((BR, 1), lambda r, n: (r, 0)),
        ],
        out_shape=[
            jax.ShapeDtypeStruct((B, N), jnp.float32),
            jax.ShapeDtypeStruct((B, 1), jnp.float32),
        ],
        scratch_shapes=[pltpu.VMEM((BR, N), jnp.float32)],
        compiler_params=pltpu.CompilerParams(
            dimension_semantics=("parallel", "arbitrary")),
    )(x, we16, bp2)

    BR2 = min(512, B)
    BK = min(2048, N)
    rb2, kb = B // BR2, N // BK
    wd16 = W_dec.astype(jnp.bfloat16)
    xhat, res = pl.pallas_call(
        functools.partial(_decode_body, kb),
        grid=(rb2, kb),
        in_specs=[
            pl.BlockSpec((BR2, BK), lambda r, k: (r, k)),
            pl.BlockSpec((BK, D), lambda r, k: (k, 0)),
            pl.BlockSpec((BR2, D), lambda r, k: (r, 0)),
            pl.BlockSpec((1, D), lambda r, k: (0, 0)),
        ],
        out_specs=[
            pl.BlockSpec((BR2, D), lambda r, k: (r, 0)),
            pl.BlockSpec((1, 1, 128), lambda r, k: (r, 0, 0)),
        ],
        out_shape=[
            jax.ShapeDtypeStruct((B, D), jnp.float32),
            jax.ShapeDtypeStruct((rb2, 1, 128), jnp.float32),
        ],
        scratch_shapes=[pltpu.VMEM((BR2, D), jnp.float32)],
        compiler_params=pltpu.CompilerParams(
            dimension_semantics=("parallel", "arbitrary")),
    )(hs, wd16, x, bp2)

    recon_loss = jnp.sum(res[:, 0, 0]) / (B * D)
    l0 = jnp.sum(cnt) / B
    return (xhat, hs, recon_loss, l0)


# speedup vs baseline: 2.7018x; 1.1044x over previous
import functools

import jax
import jax.numpy as jnp
from jax import lax
from jax.experimental import pallas as pl
from jax.experimental.pallas import tpu as pltpu

K = 32


def _encode_body(n_blocks, xb_ref, wb_ref, bp_ref, hs_ref, cnt_ref, h_acc):
    j = pl.program_id(1)
    bn = wb_ref.shape[1]
    xb = (xb_ref[...] - bp_ref[...]).astype(jnp.bfloat16)
    hblk = jnp.dot(xb, wb_ref[...], preferred_element_type=jnp.float32)
    hblk = jnp.maximum(hblk, 0.0)
    h_acc[:, pl.ds(j * bn, bn)] = hblk

    @pl.when(j == n_blocks - 1)
    def _():
        h = h_acc[...]
        br, n = h.shape
        m = jnp.max(h.reshape(br, K, n // K), axis=2)
        lo = lax.bitcast_convert_type(jnp.min(m, axis=1, keepdims=True),
                                      jnp.int32)
        hi = lax.bitcast_convert_type(jnp.max(m, axis=1, keepdims=True),
                                      jnp.int32) + 1

        chi = jnp.zeros_like(lo)

        def cond(carry):
            lo, hi, chi = carry
            return jnp.any(~((chi == K - 1) | (hi - lo <= 1)))

        def body(carry):
            lo, hi, chi = carry
            mid = lo + lax.div(hi - lo, 2)
            midf = lax.bitcast_convert_type(mid, jnp.float32)
            cnt = jnp.sum((h >= midf).astype(jnp.float32), axis=1,
                          keepdims=True).astype(jnp.int32)
            ge = cnt >= K
            return (jnp.where(ge, mid, lo), jnp.where(ge, hi, mid),
                    jnp.where(ge, chi, cnt))

        lo, hi, chi = lax.while_loop(cond, body, (lo, hi, chi))
        hif = lax.bitcast_convert_type(hi, jnp.float32)
        t = jnp.max(jnp.where(h < hif, h, 0.0), axis=1, keepdims=True)
        hs = jnp.where(h >= t, h, 0.0)
        hs_ref[...] = hs
        cnt_ref[...] = jnp.sum((hs > 0.0).astype(jnp.float32), axis=1,
                               keepdims=True)


def _decode_body(k_blocks, hs_ref, wd_ref, x_ref, bp_ref, xhat_ref, res_ref,
                 acc):
    k = pl.program_id(1)

    @pl.when(k == 0)
    def _():
        acc[...] = jnp.zeros_like(acc)

    acc[...] += jnp.dot(hs_ref[...].astype(jnp.bfloat16), wd_ref[...],
                        preferred_element_type=jnp.float32)

    @pl.when(k == k_blocks - 1)
    def _():
        xh = acc[...] + bp_ref[...]
        xhat_ref[...] = xh
        d = x_ref[...] - xh
        res_ref[...] = jnp.broadcast_to(jnp.sum(d * d), (1, 1, 128))


def kernel(x, W_enc, W_dec, b_pre):
    B, D = x.shape
    N = W_enc.shape[1]
    bp2 = b_pre.reshape(1, D)

    BR = min(128, B)
    BN = min(512, N)
    rb, nb = B // BR, N // BN
    we16 = W_enc.astype(jnp.bfloat16)
    hs, cnt = pl.pallas_call(
        functools.partial(_encode_body, nb),
        grid=(rb, nb),
        in_specs=[
            pl.BlockSpec((BR, D), lambda r, n: (r, 0)),
            pl.BlockSpec((D, BN), lambda r, n: (0, n)),
            pl.BlockSpec((1, D), lambda r, n: (0, 0)),
        ],
        out_specs=[
            pl.BlockSpec((BR, N), lambda r, n: (r, 0)),
            pl.BlockSpec((BR, 1), lambda r, n: (r, 0)),
        ],
        out_shape=[
            jax.ShapeDtypeStruct((B, N), jnp.float32),
            jax.ShapeDtypeStruct((B, 1), jnp.float32),
        ],
        scratch_shapes=[pltpu.VMEM((BR, N), jnp.float32)],
        compiler_params=pltpu.CompilerParams(
            dimension_semantics=("parallel", "arbitrary")),
    )(x, we16, bp2)

    BR2 = min(512, B)
    BK = min(2048, N)
    rb2, kb = B // BR2, N // BK
    wd16 = W_dec.astype(jnp.bfloat16)
    xhat, res = pl.pallas_call(
        functools.partial(_decode_body, kb),
        grid=(rb2, kb),
        in_specs=[
            pl.BlockSpec((BR2, BK), lambda r, k: (r, k)),
            pl.BlockSpec((BK, D), lambda r, k: (k, 0)),
            pl.BlockSpec((BR2, D), lambda r, k: (r, 0)),
            pl.BlockSpec((1, D), lambda r, k: (0, 0)),
        ],
        out_specs=[
            pl.BlockSpec((BR2, D), lambda r, k: (r, 0)),
            pl.BlockSpec((1, 1, 128), lambda r, k: (r, 0, 0)),
        ],
        out_shape=[
            jax.ShapeDtypeStruct((B, D), jnp.float32),
            jax.ShapeDtypeStruct((rb2, 1, 128), jnp.float32),
        ],
        scratch_shapes=[pltpu.VMEM((BR2, D), jnp.float32)],
        compiler_params=pltpu.CompilerParams(
            dimension_semantics=("parallel", "arbitrary")),
    )(hs, wd16, x, bp2)

    recon_loss = jnp.sum(res[:, 0, 0]) / (B * D)
    l0 = jnp.sum(cnt) / B
    return (xhat, hs, recon_loss, l0)


# speedup vs baseline: 3.1448x; 1.1640x over previous
import functools

import jax
import jax.numpy as jnp
from jax import lax
from jax.experimental import pallas as pl
from jax.experimental.pallas import tpu as pltpu

K = 32


def _encode_body(n_blocks, xb_ref, wb_ref, bp_ref, hs_ref, cnt_ref, h_acc):
    j = pl.program_id(1)
    bn = wb_ref.shape[1]
    xb = (xb_ref[...] - bp_ref[...]).astype(jnp.bfloat16)
    hblk = jnp.dot(xb, wb_ref[...], preferred_element_type=jnp.float32)
    hblk = jnp.maximum(hblk, 0.0)
    h_acc[:, pl.ds(j * bn, bn)] = hblk

    @pl.when(j == n_blocks - 1)
    def _():
        h = h_acc[...]
        br, n = h.shape
        m = jnp.max(h.reshape(br, K, n // K), axis=2)
        lo = lax.bitcast_convert_type(jnp.min(m, axis=1, keepdims=True),
                                      jnp.int32)
        hi = lax.bitcast_convert_type(jnp.max(m, axis=1, keepdims=True),
                                      jnp.int32) + 1

        chi = jnp.zeros_like(lo)

        def cond(carry):
            lo, hi, chi = carry
            return jnp.any(~((chi == K - 1) | (hi - lo <= 1)))

        def body(carry):
            lo, hi, chi = carry
            mid = lo + lax.div(hi - lo, 2)
            midf = lax.bitcast_convert_type(mid, jnp.float32)
            cnt = jnp.sum((h >= midf).astype(jnp.float32), axis=1,
                          keepdims=True).astype(jnp.int32)
            ge = cnt >= K
            return (jnp.where(ge, mid, lo), jnp.where(ge, hi, mid),
                    jnp.where(ge, chi, cnt))

        lo, hi, chi = lax.while_loop(cond, body, (lo, hi, chi))
        hif = lax.bitcast_convert_type(hi, jnp.float32)
        t = jnp.max(jnp.where(h < hif, h, 0.0), axis=1, keepdims=True)
        hs = jnp.where(h >= t, h, 0.0)
        hs_ref[...] = hs
        cnt_ref[...] = jnp.sum((hs > 0.0).astype(jnp.float32), axis=1,
                               keepdims=True)


def _decode_body(k_blocks, hs_ref, wd_ref, x_ref, bp_ref, xhat_ref, res_ref,
                 acc):
    k = pl.program_id(1)

    @pl.when(k == 0)
    def _():
        acc[...] = jnp.zeros_like(acc)

    acc[...] += jnp.dot(hs_ref[...].astype(jnp.bfloat16), wd_ref[...],
                        preferred_element_type=jnp.float32)

    @pl.when(k == k_blocks - 1)
    def _():
        xh = acc[...] + bp_ref[...]
        xhat_ref[...] = xh
        d = x_ref[...] - xh
        res_ref[...] = jnp.broadcast_to(jnp.sum(d * d), (1, 1, 128))


def kernel(x, W_enc, W_dec, b_pre):
    B, D = x.shape
    N = W_enc.shape[1]
    bp2 = b_pre.reshape(1, D)

    BR = min(128, B)
    BN = min(1024, N)
    rb, nb = B // BR, N // BN
    we16 = W_enc.astype(jnp.bfloat16)
    hs, cnt = pl.pallas_call(
        functools.partial(_encode_body, nb),
        grid=(rb, nb),
        in_specs=[
            pl.BlockSpec((BR, D), lambda r, n: (r, 0)),
            pl.BlockSpec((D, BN), lambda r, n: (0, n)),
            pl.BlockSpec((1, D), lambda r, n: (0, 0)),
        ],
        out_specs=[
            pl.BlockSpec((BR, N), lambda r, n: (r, 0)),
            pl.BlockSpec((BR, 1), lambda r, n: (r, 0)),
        ],
        out_shape=[
            jax.ShapeDtypeStruct((B, N), jnp.float32),
            jax.ShapeDtypeStruct((B, 1), jnp.float32),
        ],
        scratch_shapes=[pltpu.VMEM((BR, N), jnp.float32)],
        compiler_params=pltpu.CompilerParams(
            dimension_semantics=("parallel", "arbitrary")),
    )(x, we16, bp2)

    BR2 = min(512, B)
    BK = min(2048, N)
    rb2, kb = B // BR2, N // BK
    wd16 = W_dec.astype(jnp.bfloat16)
    xhat, res = pl.pallas_call(
        functools.partial(_decode_body, kb),
        grid=(rb2, kb),
        in_specs=[
            pl.BlockSpec((BR2, BK), lambda r, k: (r, k)),
            pl.BlockSpec((BK, D), lambda r, k: (k, 0)),
            pl.BlockSpec((BR2, D), lambda r, k: (r, 0)),
            pl.BlockSpec((1, D), lambda r, k: (0, 0)),
        ],
        out_specs=[
            pl.BlockSpec((BR2, D), lambda r, k: (r, 0)),
            pl.BlockSpec((1, 1, 128), lambda r, k: (r, 0, 0)),
        ],
        out_shape=[
            jax.ShapeDtypeStruct((B, D), jnp.float32),
            jax.ShapeDtypeStruct((rb2, 1, 128), jnp.float32),
        ],
        scratch_shapes=[pltpu.VMEM((BR2, D), jnp.float32)],
        compiler_params=pltpu.CompilerParams(
            dimension_semantics=("parallel", "arbitrary")),
    )(hs, wd16, x, bp2)

    recon_loss = jnp.sum(res[:, 0, 0]) / (B * D)
    l0 = jnp.sum(cnt) / B
    return (xhat, hs, recon_loss, l0)


# speedup vs baseline: 3.3825x; 1.0756x over previous
import functools

import jax
import jax.numpy as jnp
from jax import lax
from jax.experimental import pallas as pl
from jax.experimental.pallas import tpu as pltpu

K = 32


def _encode_body(n_blocks, xb_ref, wb_ref, bp_ref, hs_ref, cnt_ref, h_acc):
    j = pl.program_id(1)
    bn = wb_ref.shape[1]
    xb = (xb_ref[...] - bp_ref[...]).astype(jnp.bfloat16)
    hblk = jnp.dot(xb, wb_ref[...], preferred_element_type=jnp.float32)
    hblk = jnp.maximum(hblk, 0.0)
    h_acc[:, pl.ds(j * bn, bn)] = hblk

    @pl.when(j == n_blocks - 1)
    def _():
        h = h_acc[...]
        br, n = h.shape
        m = jnp.max(h.reshape(br, K, n // K), axis=2)
        lo = lax.bitcast_convert_type(jnp.min(m, axis=1, keepdims=True),
                                      jnp.int32)
        hi = lax.bitcast_convert_type(jnp.max(m, axis=1, keepdims=True),
                                      jnp.int32) + 1

        chi = jnp.zeros_like(lo)

        def cond(carry):
            lo, hi, chi = carry
            return jnp.any(~((chi == K - 1) | (hi - lo <= 1)))

        def body(carry):
            lo, hi, chi = carry
            mid = lo + lax.div(hi - lo, 2)
            midf = lax.bitcast_convert_type(mid, jnp.float32)
            cnt = jnp.sum((h >= midf).astype(jnp.float32), axis=1,
                          keepdims=True).astype(jnp.int32)
            ge = cnt >= K
            return (jnp.where(ge, mid, lo), jnp.where(ge, hi, mid),
                    jnp.where(ge, chi, cnt))

        lo, hi, chi = lax.while_loop(cond, body, (lo, hi, chi))
        hif = lax.bitcast_convert_type(hi, jnp.float32)
        t = jnp.max(jnp.where(h < hif, h, 0.0), axis=1, keepdims=True)
        hs = jnp.where(h >= t, h, 0.0)
        hs_ref[...] = hs
        cnt_ref[...] = jnp.sum((hs > 0.0).astype(jnp.float32), axis=1,
                               keepdims=True)


def _decode_body(k_blocks, hs_ref, wd_ref, x_ref, bp_ref, xhat_ref, res_ref,
                 acc):
    k = pl.program_id(1)

    @pl.when(k == 0)
    def _():
        acc[...] = jnp.zeros_like(acc)

    acc[...] += jnp.dot(hs_ref[...].astype(jnp.bfloat16), wd_ref[...],
                        preferred_element_type=jnp.float32)

    @pl.when(k == k_blocks - 1)
    def _():
        xh = acc[...] + bp_ref[...]
        xhat_ref[...] = xh
        d = x_ref[...] - xh
        res_ref[...] = jnp.broadcast_to(jnp.sum(d * d), (1, 1, 128))


def kernel(x, W_enc, W_dec, b_pre):
    B, D = x.shape
    N = W_enc.shape[1]
    bp2 = b_pre.reshape(1, D)

    BR = min(128, B)
    BN = min(2048, N)
    rb, nb = B // BR, N // BN
    we16 = W_enc.astype(jnp.bfloat16)
    hs, cnt = pl.pallas_call(
        functools.partial(_encode_body, nb),
        grid=(rb, nb),
        in_specs=[
            pl.BlockSpec((BR, D), lambda r, n: (r, 0)),
            pl.BlockSpec((D, BN), lambda r, n: (0, n)),
            pl.BlockSpec((1, D), lambda r, n: (0, 0)),
        ],
        out_specs=[
            pl.BlockSpec((BR, N), lambda r, n: (r, 0)),
            pl.BlockSpec((BR, 1), lambda r, n: (r, 0)),
        ],
        out_shape=[
            jax.ShapeDtypeStruct((B, N), jnp.float32),
            jax.ShapeDtypeStruct((B, 1), jnp.float32),
        ],
        scratch_shapes=[pltpu.VMEM((BR, N), jnp.float32)],
        compiler_params=pltpu.CompilerParams(
            dimension_semantics=("parallel", "arbitrary")),
    )(x, we16, bp2)

    BR2 = min(512, B)
    BK = min(2048, N)
    rb2, kb = B // BR2, N // BK
    wd16 = W_dec.astype(jnp.bfloat16)
    xhat, res = pl.pallas_call(
        functools.partial(_decode_body, kb),
        grid=(rb2, kb),
        in_specs=[
            pl.BlockSpec((BR2, BK), lambda r, k: (r, k)),
            pl.BlockSpec((BK, D), lambda r, k: (k, 0)),
            pl.BlockSpec((BR2, D), lambda r, k: (r, 0)),
            pl.BlockSpec((1, D), lambda r, k: (0, 0)),
        ],
        out_specs=[
            pl.BlockSpec((BR2, D), lambda r, k: (r, 0)),
            pl.BlockSpec((1, 1, 128), lambda r, k: (r, 0, 0)),
        ],
        out_shape=[
            jax.ShapeDtypeStruct((B, D), jnp.float32),
            jax.ShapeDtypeStruct((rb2, 1, 128), jnp.float32),
        ],
        scratch_shapes=[pltpu.VMEM((BR2, D), jnp.float32)],
        compiler_params=pltpu.CompilerParams(
            dimension_semantics=("parallel", "arbitrary")),
    )(hs, wd16, x, bp2)

    recon_loss = jnp.sum(res[:, 0, 0]) / (B * D)
    l0 = jnp.sum(cnt) / B
    return (xhat, hs, recon_loss, l0)
